# baseline (speedup 1.0000x reference)
import functools

import jax
import jax.numpy as jnp
from jax.experimental import pallas as pl
from jax.experimental.pallas import tpu as pltpu

N, D, E, TOP_K = 8192, 2048, 8, 2
BLK = 256
M = TOP_K * N + E * BLK
NB = M // BLK
GB = 1024


def _gate_body(x_ref, gw_ref, gb_ref, itop_ref, wtop_ref):
    x = x_ref[...]
    logits = jax.lax.dot_general(
        x, gw_ref[...], (((1,), (1,)), ((), ())),
        preferred_element_type=jnp.float32) + gb_ref[...]
    probs = jax.nn.softmax(logits, axis=-1)
    cols = jax.lax.broadcasted_iota(jnp.int32, probs.shape, 1)
    i1 = jnp.argmax(probs, axis=-1, keepdims=True)
    p1 = jnp.max(probs, axis=-1, keepdims=True)
    pm = jnp.where(cols == i1, -jnp.inf, probs)
    i2 = jnp.argmax(pm, axis=-1, keepdims=True)
    p2 = jnp.max(pm, axis=-1, keepdims=True)
    itop_ref[...] = jnp.concatenate([i1, i2], axis=1)
    wtop_ref[...] = jnp.concatenate([p1, p2], axis=1)


def _gate(x, gate_w, gate_b):
    return pl.pallas_call(
        _gate_body,
        grid=(N // GB,),
        in_specs=[
            pl.BlockSpec((GB, D), lambda n: (n, 0)),
            pl.BlockSpec((E, D), lambda n: (0, 0)),
            pl.BlockSpec((1, E), lambda n: (0, 0)),
        ],
        out_specs=[
            pl.BlockSpec((GB, TOP_K), lambda n: (n, 0)),
            pl.BlockSpec((GB, TOP_K), lambda n: (n, 0)),
        ],
        out_shape=[
            jax.ShapeDtypeStruct((N, TOP_K), jnp.int32),
            jax.ShapeDtypeStruct((N, TOP_K), jnp.float32),
        ],
    )(x, gate_w, gate_b.reshape(1, E))


def _route(itop, wtop):
    flat_e = itop.reshape(-1)
    tok = jax.lax.iota(jnp.int32, TOP_K * N) // TOP_K
    oh = (flat_e[:, None] == jnp.arange(E, dtype=jnp.int32)[None, :])
    csum = jnp.cumsum(oh.astype(jnp.int32), axis=0)
    rank = jnp.take_along_axis(csum - 1, flat_e[:, None], axis=1)[:, 0]
    cnt = csum[-1]
    padded = ((cnt + BLK - 1) // BLK) * BLK
    off = jnp.concatenate([jnp.zeros((1,), jnp.int32),
                           jnp.cumsum(padded)[:-1].astype(jnp.int32)])
    pos = off[flat_e] + rank
    row_token = jnp.zeros((M,), jnp.int32).at[pos].set(tok)
    row_weight = jnp.zeros((M,), jnp.float32).at[pos].set(wtop.reshape(-1))
    block_expert = jnp.clip(
        jnp.searchsorted(off, jnp.arange(NB, dtype=jnp.int32) * BLK,
                         side="right") - 1, 0, E - 1).astype(jnp.int32)
    pos1 = pos[0::TOP_K]
    pos2 = pos[1::TOP_K]
    return row_token, row_weight, block_expert, pos1, pos2


def _mm_body(be_ref, xg_ref, w_ref, b_ref, rw_ref, yg_ref):
    xg = xg_ref[...]
    y = jax.lax.dot_general(xg, w_ref[0], (((1,), (1,)), ((), ())),
                            preferred_element_type=jnp.float32)
    yg_ref[...] = (y + b_ref[0]) * rw_ref[...]


def _grouped_mm(xg, expert_w, expert_b, row_weight, block_expert):
    grid_spec = pltpu.PrefetchScalarGridSpec(
        num_scalar_prefetch=1,
        grid=(NB,),
        in_specs=[
            pl.BlockSpec((BLK, D), lambda j, be: (j, 0)),
            pl.BlockSpec((1, D, D), lambda j, be: (be[j], 0, 0)),
            pl.BlockSpec((1, 1, D), lambda j, be: (be[j], 0, 0)),
            pl.BlockSpec((BLK, 1), lambda j, be: (j, 0)),
        ],
        out_specs=pl.BlockSpec((BLK, D), lambda j, be: (j, 0)),
    )
    return pl.pallas_call(
        _mm_body,
        grid_spec=grid_spec,
        out_shape=jax.ShapeDtypeStruct((M, D), jnp.float32),
    )(block_expert, xg, expert_w, expert_b.reshape(E, 1, D),
      row_weight.reshape(M, 1))


def _gather_rows(x, row_token):
    return x[row_token]


def _combine(yg, pos1, pos2):
    return yg[pos1] + yg[pos2]


@jax.jit
def kernel(x, gate_w, gate_b, expert_w, expert_b):
    itop, wtop = _gate(x, gate_w, gate_b)
    row_token, row_weight, block_expert, pos1, pos2 = _route(itop, wtop)
    xg = _gather_rows(x, row_token)
    yg = _grouped_mm(xg, expert_w, expert_b, row_weight, block_expert)
    return _combine(yg, pos1, pos2)
